# baseline (speedup 1.0000x reference)
import jax
import jax.numpy as jnp
from jax import lax
from jax.experimental import pallas as pl
from jax.experimental.pallas import tpu as pltpu
from jax.experimental.pallas import tpu_sc as plsc

D_VOCAB = 100000
N_ROWS = 1024
N_TOK = 2 * N_ROWS
CH = 16
CH_LOG = 4
NCHUNK = D_VOCAB // CH
NC = 2
NS = 16
L = 16
NW = NC * NS
QFULL = NCHUNK // NW
NEXTRA = NCHUNK - QFULL * NW
LISTCAP = N_ROWS + L
RING = 6


def _body(x_hbm, out_hbm, idx_v, cidA, addrA, cidB, addrB, buf, sem):
    wid = lax.axis_index("s") * NC + lax.axis_index("c")

    pltpu.sync_copy(x_hbm, idx_v)

    zeros16 = jnp.zeros((L,), jnp.float32)
    ones16 = jnp.ones((L,), jnp.float32)
    iota16 = lax.iota(jnp.int32, L)
    even16 = (iota16 & 1) == 0
    half16 = iota16 >> 1

    def _scan(q, carry):
        ca, cb = carry
        tvec = idx_v[pl.ds(q * L, L)]
        cid = tvec >> CH_LOG
        mine = (cid & (NW - 1)) == wid
        bvec = q * 8 + half16
        addr = ((tvec & (CH - 1)) << 10) | bvec
        mA = mine & even16
        mB = mine & jnp.logical_not(even16)
        plsc.store_compressed(cidA.at[pl.ds(ca, L)], cid, mask=mA)
        plsc.store_compressed(addrA.at[pl.ds(ca, L)], addr, mask=mA)
        plsc.store_compressed(cidB.at[pl.ds(cb, L)], cid, mask=mB)
        plsc.store_compressed(addrB.at[pl.ds(cb, L)], addr, mask=mB)
        ca = ca + jnp.sum(mA.astype(jnp.int32))
        cb = cb + jnp.sum(mB.astype(jnp.int32))
        return ca, cb

    cntA, cntB = lax.fori_loop(0, N_TOK // L, _scan, (jnp.int32(0),
                                                      jnp.int32(0)))
    nvA = (cntA + L - 1) >> 4
    nvB = (cntB + L - 1) >> 4

    def _pass(j, row_off, value):
        def _one(cid_ref, addr_ref, cnt, nv, lane_sel):
            def _vec(i, carry):
                cv = cid_ref[pl.ds(i * L, L)]
                av = addr_ref[pl.ds(i * L, L)]
                valid = (i * L + iota16) < cnt
                m = (cv == j) & valid & lane_sel
                rows = row_off + (av >> 10)
                cols = av & (N_ROWS - 1)
                if value == 0.0:
                    plsc.store_scatter(buf, [rows, cols], zeros16, mask=m)
                else:
                    plsc.addupdate_scatter(buf, [rows, cols], ones16,
                                           mask=m)
                return carry

            lax.fori_loop(0, nv, _vec, 0)

        true16 = iota16 >= 0
        _one(cidA, addrA, cntA, nvA, true16)
        _one(cidB, addrB, cntB, nvB, true16)

    def _chunk(jj, carry):
        j = wid + NW * jj
        h = (jj - (jj // RING) * RING) * CH

        @pl.when(jj < RING)
        def _():
            def _zrow(i, carry):
                for k in range(N_ROWS // L):
                    buf[i, pl.ds(k * L, L)] = zeros16
                return carry

            lax.fori_loop(h, h + CH, _zrow, 0)

        @pl.when(jj >= RING)
        def _():
            pltpu.make_async_copy(
                buf.at[pl.ds(h, CH)], out_hbm.at[pl.ds(0, CH)], sem).wait()
            _pass(j - RING * NW, h, 0.0)

        _pass(j, h, 1.0)
        pltpu.async_copy(
            buf.at[pl.ds(h, CH)], out_hbm.at[pl.ds(j * CH, CH)], sem)
        return carry

    lax.fori_loop(0, QFULL, _chunk, 0)

    for _ in range(RING):
        pltpu.make_async_copy(
            buf.at[pl.ds(0, CH)], out_hbm.at[pl.ds(0, CH)], sem).wait()

    @pl.when(wid < NEXTRA)
    def _():
        j = QFULL * NW + wid
        h = (QFULL % RING) * CH
        _pass(wid + NW * (QFULL - RING), h, 0.0)
        _pass(j, h, 1.0)
        pltpu.sync_copy(buf.at[pl.ds(h, CH)],
                        out_hbm.at[pl.ds(j * CH, CH)])


@jax.jit
def _embed(x_flat):
    mesh = plsc.VectorSubcoreMesh(
        core_axis_name="c", subcore_axis_name="s", num_cores=NC,
        num_subcores=NS)
    f = pl.kernel(
        _body,
        out_type=jax.ShapeDtypeStruct((D_VOCAB, N_ROWS), jnp.float32),
        mesh=mesh,
        scratch_types=[
            pltpu.VMEM((N_TOK,), jnp.int32),
            pltpu.VMEM((LISTCAP,), jnp.int32),
            pltpu.VMEM((LISTCAP,), jnp.int32),
            pltpu.VMEM((LISTCAP,), jnp.int32),
            pltpu.VMEM((LISTCAP,), jnp.int32),
            pltpu.VMEM((RING * CH, N_ROWS), jnp.float32),
            pltpu.SemaphoreType.DMA,
        ],
        compiler_params=pltpu.CompilerParams(needs_layout_passes=False),
    )
    return f(x_flat)


def kernel(x, W_E):
    del W_E
    out_t = _embed(x.reshape(-1).astype(jnp.int32))
    return out_t.T[:, None, :]
